# Initial kernel scaffold; baseline (speedup 1.0000x reference)
import jax
import jax.numpy as jnp
from jax.experimental import pallas as pl


def kernel(inputs, table, pos):
    raise NotImplementedError("write your pallas kernel here")


# speedup vs baseline: 1.2934x; 1.2934x over previous
import functools

import jax
import jax.numpy as jnp
from jax import lax
from jax.experimental import pallas as pl
from jax.experimental.pallas import tpu as pltpu
from jax.experimental.pallas import tpu_sc as plsc

_D = 32
_NC = 2
_NS = 16
_NW = _NC * _NS
_LANES = 16


def _make_kernel(n_rows, chunk):
    rpw = n_rows // _NW
    nch = rpw // chunk

    mesh = plsc.VectorSubcoreMesh(core_axis_name="c", subcore_axis_name="s")

    @functools.partial(
        pl.kernel,
        mesh=mesh,
        compiler_params=pltpu.CompilerParams(use_tc_tiling_on_sc=False),
        out_type=jax.ShapeDtypeStruct((n_rows, _D), jnp.float32),
        scratch_types=[
            pltpu.VMEM((chunk,), jnp.int32),
            pltpu.VMEM((chunk, _D), jnp.float32),
            pltpu.VMEM((chunk, _D), jnp.float32),
            pltpu.SemaphoreType.DMA,
        ],
    )
    def k(idx_hbm, table_hbm, ptile_hbm, out_hbm, idx_v, rows_v, ptile_v, sem):
        wid = lax.axis_index("s") * _NC + lax.axis_index("c")
        base = wid * rpw
        pltpu.sync_copy(ptile_hbm, ptile_v)

        def chunk_body(g, carry):
            off = base + g * chunk
            pltpu.sync_copy(idx_hbm.at[pl.ds(off, chunk)], idx_v)
            pltpu.async_copy(table_hbm.at[idx_v], rows_v, sem).wait()

            def row_body(r, c):
                rows_v[r, pl.ds(0, _LANES)] = (
                    rows_v[r, pl.ds(0, _LANES)] + ptile_v[r, pl.ds(0, _LANES)]
                )
                rows_v[r, pl.ds(_LANES, _LANES)] = (
                    rows_v[r, pl.ds(_LANES, _LANES)]
                    + ptile_v[r, pl.ds(_LANES, _LANES)]
                )
                return c

            lax.fori_loop(0, chunk, row_body, 0)
            pltpu.sync_copy(rows_v, out_hbm.at[pl.ds(off, chunk)])
            return carry

        lax.fori_loop(0, nch, chunk_body, 0)

    return k


def kernel(inputs, table, pos):
    b, l = inputs.shape
    n_rows = b * l
    idx = inputs.reshape(n_rows).astype(jnp.int32)
    chunk = 4 * l
    ptile = jnp.tile(pos, (chunk // l, 1))
    out = _make_kernel(n_rows, chunk)(idx, table, ptile)
    return out.reshape(b, l, _D)


# speedup vs baseline: 1.2999x; 1.0051x over previous
import functools

import jax
import jax.numpy as jnp
from jax import lax
from jax.experimental import pallas as pl
from jax.experimental.pallas import tpu as pltpu
from jax.experimental.pallas import tpu_sc as plsc

_D = 32
_NC = 2
_NS = 16
_NW = _NC * _NS
_LANES = 16


def _make_kernel(n_rows, chunk):
    rpw = n_rows // _NW
    nch = rpw // chunk

    mesh = plsc.VectorSubcoreMesh(core_axis_name="c", subcore_axis_name="s")

    @functools.partial(
        pl.kernel,
        mesh=mesh,
        compiler_params=pltpu.CompilerParams(use_tc_tiling_on_sc=False),
        out_type=jax.ShapeDtypeStruct((n_rows, _D), jnp.float32),
        scratch_types=[
            pltpu.VMEM((chunk,), jnp.int32),
            pltpu.VMEM((chunk, _D), jnp.float32),
            pltpu.VMEM((chunk, _D), jnp.float32),
            pltpu.SemaphoreType.DMA,
        ],
    )
    def k(idx_hbm, table_hbm, ptile_hbm, out_hbm, idx_v, rows_v, ptile_v, sem):
        wid = lax.axis_index("s") * _NC + lax.axis_index("c")
        base = wid * rpw
        pltpu.sync_copy(ptile_hbm, ptile_v)

        def chunk_body(g, carry):
            off = base + g * chunk
            pltpu.sync_copy(idx_hbm.at[pl.ds(off, chunk)], idx_v)
            pltpu.sync_copy(ptile_hbm, rows_v)
            pltpu.async_copy(table_hbm.at[idx_v], rows_v, sem, add=True).wait()
            pltpu.sync_copy(rows_v, out_hbm.at[pl.ds(off, chunk)])
            return carry

        lax.fori_loop(0, nch, chunk_body, 0)

    return k


def kernel(inputs, table, pos):
    b, l = inputs.shape
    n_rows = b * l
    idx = inputs.reshape(n_rows).astype(jnp.int32)
    chunk = 4 * l
    ptile = jnp.tile(pos, (chunk // l, 1))
    out = _make_kernel(n_rows, chunk)(idx, table, ptile)
    return out.reshape(b, l, _D)


# speedup vs baseline: 1.3871x; 1.0670x over previous
import functools

import jax
import jax.numpy as jnp
from jax import lax
from jax.experimental import pallas as pl
from jax.experimental.pallas import tpu as pltpu
from jax.experimental.pallas import tpu_sc as plsc

_D = 32
_NC = 2
_NS = 16
_NW = _NC * _NS


def _make_kernel(n_rows, chunk):
    rpw = n_rows // _NW
    nch = rpw // chunk

    mesh = plsc.VectorSubcoreMesh(core_axis_name="c", subcore_axis_name="s")

    @functools.partial(
        pl.kernel,
        mesh=mesh,
        compiler_params=pltpu.CompilerParams(use_tc_tiling_on_sc=False),
        out_type=jax.ShapeDtypeStruct((n_rows, _D), jnp.float32),
        scratch_types=[
            pltpu.VMEM((chunk,), jnp.int32),
            pltpu.VMEM((chunk, _D), jnp.float32),
            pltpu.SemaphoreType.DMA,
        ],
    )
    def k(idx_hbm, table_hbm, ptile_hbm, out_hbm, idx_v, rows_v, sem):
        wid = lax.axis_index("s") * _NC + lax.axis_index("c")
        base = wid * rpw

        def chunk_body(g, carry):
            off = base + g * chunk
            pltpu.sync_copy(idx_hbm.at[pl.ds(off, chunk)], idx_v)
            pltpu.sync_copy(ptile_hbm, rows_v)
            pltpu.async_copy(table_hbm.at[idx_v], rows_v, sem, add=True).wait()
            pltpu.sync_copy(rows_v, out_hbm.at[pl.ds(off, chunk)])
            return carry

        lax.fori_loop(0, nch, chunk_body, 0)

    return k


def kernel(inputs, table, pos):
    b, l = inputs.shape
    n_rows = b * l
    idx = inputs.reshape(n_rows).astype(jnp.int32)
    chunk = 16 * l
    ptile = jnp.tile(pos, (chunk // l, 1))
    out = _make_kernel(n_rows, chunk)(idx, table, ptile)
    return out.reshape(b, l, _D)
